# Initial kernel scaffold; baseline (speedup 1.0000x reference)
import jax
import jax.numpy as jnp
from jax.experimental import pallas as pl


def kernel(x, edge_index, params):
    raise NotImplementedError("write your pallas kernel here")


# speedup vs baseline: 2.6825x; 2.6825x over previous
import functools
import jax
import jax.numpy as jnp
from jax.experimental import pallas as pl
from jax.experimental.pallas import tpu as pltpu

HID = 128
NUM_CHOICES = 7
NUM_SELECTED = 4
NUM_COLUMNS = 8

ROW_BLK = 2000


def _mm_bias_body(h_ref, w_ref, b_ref, o_ref):
    o_ref[...] = (
        jnp.dot(h_ref[...], w_ref[...], preferred_element_type=jnp.float32)
        + b_ref[...]
    )


def _mm_bias(h, W, b):
    N, K = h.shape
    grid = (N // ROW_BLK,)
    return pl.pallas_call(
        _mm_bias_body,
        grid=grid,
        in_specs=[
            pl.BlockSpec((ROW_BLK, K), lambda i: (i, 0)),
            pl.BlockSpec((K, 128), lambda i: (0, 0)),
            pl.BlockSpec((1, 128), lambda i: (0, 0)),
        ],
        out_specs=pl.BlockSpec((ROW_BLK, 128), lambda i: (i, 0)),
        out_shape=jax.ShapeDtypeStruct((N, 128), jnp.float32),
    )(h, W, b.reshape(1, 128))


def _combine_body(slope, num_ref, den_ref, b_ref, o_ref):
    h = num_ref[...] / (den_ref[...] + 1e-16) + b_ref[...]
    if slope is not None:
        h = jnp.maximum(h, slope * h)
    o_ref[...] = h


def _combine(num, den, bias, slope):
    N = num.shape[0]
    return pl.pallas_call(
        functools.partial(_combine_body, slope),
        grid=(N // ROW_BLK,),
        in_specs=[
            pl.BlockSpec((ROW_BLK, 128), lambda i: (i, 0)),
            pl.BlockSpec((ROW_BLK, 1), lambda i: (i, 0)),
            pl.BlockSpec((1, 128), lambda i: (0, 0)),
        ],
        out_specs=pl.BlockSpec((ROW_BLK, 128), lambda i: (i, 0)),
        out_shape=jax.ShapeDtypeStruct((N, 128), jnp.float32),
    )(num, den, bias.reshape(1, 128))


def _value_body(num_ref, den_ref, b_ref, w_ref, fcb_ref, o_ref):
    h = num_ref[...] / (den_ref[...] + 1e-16) + b_ref[...]
    v = jnp.dot(h, w_ref[...], preferred_element_type=jnp.float32)
    o_ref[...] = v + fcb_ref[0, 0]


def _value_head(num, den, bias, fc_W, fc_b):
    N = num.shape[0]
    fc_pad = jnp.pad(fc_W, ((0, 0), (0, 127)))
    return pl.pallas_call(
        _value_body,
        grid=(N // ROW_BLK,),
        in_specs=[
            pl.BlockSpec((ROW_BLK, 128), lambda i: (i, 0)),
            pl.BlockSpec((ROW_BLK, 1), lambda i: (i, 0)),
            pl.BlockSpec((1, 128), lambda i: (0, 0)),
            pl.BlockSpec((128, 128), lambda i: (0, 0)),
            pl.BlockSpec((1, 1), lambda i: (0, 0), memory_space=pltpu.SMEM),
        ],
        out_specs=pl.BlockSpec((ROW_BLK, 128), lambda i: (i, 0)),
        out_shape=jax.ShapeDtypeStruct((N, 128), jnp.float32),
    )(num, den, bias.reshape(1, 128), fc_pad, fc_b.reshape(1, 1))


def _sample_body(lg_ref, g_ref, act_ref, sp_ref):
    col = jax.lax.broadcasted_iota(jnp.int32, (16, 128), 1)
    valid = col < NUM_CHOICES
    lg = jnp.where(valid, lg_ref[...], -jnp.inf)
    m = jnp.max(lg, axis=1, keepdims=True)
    ex = jnp.where(valid, jnp.exp(lg - m), 0.0)
    probs = ex / jnp.sum(ex, axis=1, keepdims=True)
    score = jnp.where(valid, jnp.log(probs + 1e-12) + g_ref[...], -jnp.inf)

    actions = []
    for _ in range(NUM_SELECTED):
        mx = jnp.max(score, axis=1, keepdims=True)
        idx = jnp.min(jnp.where(score == mx, col, 128), axis=1, keepdims=True)
        actions.append(idx)
        score = jnp.where(col == idx, -jnp.inf, score)
    act = jnp.concatenate(actions, axis=1)

    row = jax.lax.broadcasted_iota(jnp.int32, (16, 128), 0)
    srow = col
    probs_p = jnp.where(col < NUM_CHOICES, probs, 0.0)
    outs = []
    for j in range(NUM_SELECTED):
        aj = act[:, j : j + 1]
        G = jnp.where(
            ((srow // 8) == (row // 8)) & ((srow % 8) == aj) & (srow < 16), 1.0, 0.0
        )
        Gp = G[:, :16]
        mmj = jnp.dot(Gp, probs_p[:16, :], preferred_element_type=jnp.float32)
        outs.append(mmj[:, j : j + 1])
    sp = jnp.concatenate(outs, axis=1)

    act_ref[...] = jnp.pad(act, ((0, 0), (0, 124)))
    sp_ref[...] = jnp.pad(sp, ((0, 0), (0, 124)))


def _sample(logits16, g16):
    return pl.pallas_call(
        _sample_body,
        out_shape=[
            jax.ShapeDtypeStruct((16, 128), jnp.int32),
            jax.ShapeDtypeStruct((16, 128), jnp.float32),
        ],
    )(logits16, g16)


def _edge_phase(xl, xr, att, src, dst, N):
    alpha = jnp.einsum("ed,d->e", jax.nn.leaky_relu(xl[src] + xr[dst], 0.2), att)
    w = jnp.exp(alpha)
    num = jax.ops.segment_sum(w[:, None] * xl[src], dst, num_segments=N)
    den = jax.ops.segment_sum(w, dst, num_segments=N)
    return num, den[:, None]


def kernel(x, edge_index, params):
    B, Nn, D = x.shape
    N = B * Nn
    src, dst = edge_index[0], edge_index[1]
    h0 = x.reshape(N, D)

    pa1 = params["a1"]
    xl = _mm_bias(h0, pa1["Wl"], pa1["bl"])
    xr = _mm_bias(h0, pa1["Wr"], pa1["br"])
    num, den = _edge_phase(xl, xr, pa1["att"][0], src, dst, N)
    h_a = _combine(num, den, pa1["bias"], 0.01)

    pa2 = params["a2"]
    Wl2 = jnp.pad(pa2["Wl"], ((0, 0), (0, 128 - NUM_CHOICES)))
    Wr2 = jnp.pad(pa2["Wr"], ((0, 0), (0, 128 - NUM_CHOICES)))
    bl2 = jnp.pad(pa2["bl"], (0, 128 - NUM_CHOICES))
    br2 = jnp.pad(pa2["br"], (0, 128 - NUM_CHOICES))
    xl2 = _mm_bias(h_a, Wl2, bl2)[:, :NUM_CHOICES]
    xr2 = _mm_bias(h_a, Wr2, br2)[:, :NUM_CHOICES]
    alpha2 = jnp.einsum(
        "ed,d->e", jax.nn.leaky_relu(xl2[src] + xr2[dst], 0.2), pa2["att"][0]
    )
    w2 = jnp.exp(alpha2)
    num2 = jax.ops.segment_sum(w2[:, None] * xl2[src], dst, num_segments=N)
    den2 = jax.ops.segment_sum(w2, dst, num_segments=N)
    logits = num2 / (den2[:, None] + 1e-16) + pa2["bias"][None, :]

    lg3 = logits.reshape(B, Nn, NUM_CHOICES)[:, :NUM_COLUMNS, :]
    u = jax.random.uniform(
        jax.random.key(123), (B, NUM_COLUMNS, NUM_CHOICES), minval=1e-9, maxval=1.0
    )
    g = -jnp.log(-jnp.log(u))
    lg16 = jnp.pad(lg3.reshape(16, NUM_CHOICES), ((0, 0), (0, 121)))
    g16 = jnp.pad(g.reshape(16, NUM_CHOICES), ((0, 0), (0, 121)))
    act_p, sp_p = _sample(lg16, g16)
    action = act_p[:, :NUM_SELECTED].reshape(B, NUM_COLUMNS, NUM_SELECTED)
    selected_probs = sp_p[:, :NUM_SELECTED].reshape(B, NUM_COLUMNS, NUM_SELECTED)

    pc1 = params["c1"]
    cl = _mm_bias(h0, pc1["Wl"], pc1["bl"])
    cr = _mm_bias(h0, pc1["Wr"], pc1["br"])
    numc, denc = _edge_phase(cl, cr, pc1["att"][0], src, dst, N)
    h_c = _combine(numc, denc, pc1["bias"], 0.01)

    pc2 = params["c2"]
    cl2 = _mm_bias(h_c, pc2["Wl"], pc2["bl"])
    cr2 = _mm_bias(h_c, pc2["Wr"], pc2["br"])
    numc2, denc2 = _edge_phase(cl2, cr2, pc2["att"][0], src, dst, N)
    vfull = _value_head(numc2, denc2, pc2["bias"], params["fc_W"], params["fc_b"])
    value = vfull[:, :1].reshape(B, Nn, 1)

    return action, selected_probs, value


# speedup vs baseline: 6.3999x; 2.3858x over previous
import functools
import jax
import jax.numpy as jnp
from jax import lax
from jax.experimental import pallas as pl
from jax.experimental.pallas import tpu as pltpu
from jax.experimental.pallas import tpu_sc as plsc

HID = 128
NUM_CHOICES = 7
NUM_SELECTED = 4
NUM_COLUMNS = 8

ROW_BLK = 2000

NTOT = 20000
ETOT = 320000
NC, NS = 2, 16
NW = NC * NS
NPS = NTOT // NS

CH = 400
EPW_A = ETOT // NW
EPS_B = ETOT // NS
EPC_B2 = ETOT // NC


def _mm_bias_body(h_ref, w_ref, b_ref, o_ref):
    o_ref[...] = (
        jnp.dot(h_ref[...], w_ref[...], preferred_element_type=jnp.float32)
        + b_ref[...]
    )


def _mm_split_body(h_ref, w_ref, b_ref, o_ref):
    o_ref[...] = (
        jnp.dot(h_ref[...], w_ref[0], preferred_element_type=jnp.float32)
        + b_ref[0]
    )


def _mm_bias_split(h, W, b):
    N = h.shape[0]
    Ws = jnp.stack([W[:, :64], W[:, 64:]])
    bs = jnp.stack([b[:64], b[64:]]).reshape(2, 1, 64)
    return pl.pallas_call(
        _mm_split_body,
        grid=(N // ROW_BLK, 2),
        in_specs=[
            pl.BlockSpec((ROW_BLK, 128), lambda i, c: (i, 0)),
            pl.BlockSpec((1, 128, 64), lambda i, c: (c, 0, 0)),
            pl.BlockSpec((1, 1, 64), lambda i, c: (c, 0, 0)),
        ],
        out_specs=pl.BlockSpec(
            (ROW_BLK, 64), lambda i, c: (c * (NTOT // ROW_BLK) + i, 0)
        ),
        out_shape=jax.ShapeDtypeStruct((2 * N, 64), jnp.float32),
    )(h, Ws, bs)


def _mm_bias16(h, W16, b16):
    N = h.shape[0]
    return pl.pallas_call(
        _mm_bias_body,
        grid=(N // ROW_BLK,),
        in_specs=[
            pl.BlockSpec((ROW_BLK, 128), lambda i: (i, 0)),
            pl.BlockSpec((128, 16), lambda i: (0, 0)),
            pl.BlockSpec((1, 16), lambda i: (0, 0)),
        ],
        out_specs=pl.BlockSpec((ROW_BLK, 16), lambda i: (i, 0)),
        out_shape=jax.ShapeDtypeStruct((N, 16), jnp.float32),
    )(h, W16, b16.reshape(1, 16))


def _combine_body(slope, num_ref, den_ref, b_ref, o_ref):
    h = jnp.concatenate([num_ref[0], num_ref[1]], axis=1)
    h = h / (den_ref[...] + 1e-16) + b_ref[...]
    if slope is not None:
        h = jnp.maximum(h, slope * h)
    o_ref[...] = h


def _combine(num, den, bias, slope):
    N = num.shape[1]
    return pl.pallas_call(
        functools.partial(_combine_body, slope),
        grid=(N // ROW_BLK,),
        in_specs=[
            pl.BlockSpec((2, ROW_BLK, 64), lambda i: (0, i, 0)),
            pl.BlockSpec((ROW_BLK, 1), lambda i: (i, 0)),
            pl.BlockSpec((1, 128), lambda i: (0, 0)),
        ],
        out_specs=pl.BlockSpec((ROW_BLK, 128), lambda i: (i, 0)),
        out_shape=jax.ShapeDtypeStruct((N, 128), jnp.float32),
    )(num, den, bias.reshape(1, 128))


def _value_body(num_ref, den_ref, b_ref, w_ref, fcb_ref, o_ref):
    h = jnp.concatenate([num_ref[0], num_ref[1]], axis=1)
    h = h / (den_ref[...] + 1e-16) + b_ref[...]
    v = jnp.dot(h, w_ref[...], preferred_element_type=jnp.float32)
    o_ref[...] = v + fcb_ref[0, 0]


def _value_head(num, den, bias, fc_W, fc_b):
    N = num.shape[1]
    fc_pad = jnp.pad(fc_W, ((0, 0), (0, 127)))
    return pl.pallas_call(
        _value_body,
        grid=(N // ROW_BLK,),
        in_specs=[
            pl.BlockSpec((2, ROW_BLK, 64), lambda i: (0, i, 0)),
            pl.BlockSpec((ROW_BLK, 1), lambda i: (i, 0)),
            pl.BlockSpec((1, 128), lambda i: (0, 0)),
            pl.BlockSpec((128, 128), lambda i: (0, 0)),
            pl.BlockSpec((1, 1), lambda i: (0, 0), memory_space=pltpu.SMEM),
        ],
        out_specs=pl.BlockSpec((ROW_BLK, 128), lambda i: (i, 0)),
        out_shape=jax.ShapeDtypeStruct((N, 128), jnp.float32),
    )(num, den, bias.reshape(1, 128), fc_pad, fc_b.reshape(1, 1))


def _passa2_body(xl_ref, xr_ref, att_ref, src_ref, dst_ref, w_ref,
                 si_v, di_v, xls_v, xrs_v, att_v, w_v, fold_v, sem):
    c = lax.axis_index("c")
    s = lax.axis_index("s")
    wid = s * NC + c
    pltpu.sync_copy(att_ref, att_v)

    def chunk(i, carry):
        base = wid * EPW_A + i * CH
        pltpu.sync_copy(src_ref.at[pl.ds(base, CH)], si_v)
        pltpu.sync_copy(dst_ref.at[pl.ds(base, CH)], di_v)
        d1 = pltpu.async_copy(xl_ref.at[si_v], xls_v, sem)
        d2 = pltpu.async_copy(xr_ref.at[di_v], xrs_v, sem)
        d1.wait()
        d2.wait()

        lane = lax.iota(jnp.int32, 16)
        fold_v[pl.ds(16, 16)] = jnp.zeros((16,), jnp.float32)

        def grp(t, carry2):
            wvec = jnp.zeros((16,), jnp.float32)
            for j in range(16):
                e = t * 16 + j
                v = xls_v[e, :] + xrs_v[e, :]
                v = jnp.maximum(v, 0.2 * v)
                acc = v * att_v[...]
                for sh in (8, 4, 2, 1):
                    fold_v[pl.ds(0, 16)] = acc
                    acc = fold_v[pl.ds(0, 16)] + fold_v[pl.ds(sh, 16)]
                wvec = jnp.where(lane == j, acc[0], wvec)
            w_v[pl.ds(t * 16, 16)] = jnp.exp(wvec)
            return carry2
        lax.fori_loop(0, CH // 16, grp, 0)

        pltpu.sync_copy(w_v, w_ref.at[pl.ds(base, CH)])
        return carry
    lax.fori_loop(0, EPW_A // CH, chunk, 0)


def _passa2(xl16, xr16, att16, src, dst):
    return pl.kernel(
        _passa2_body,
        out_type=jax.ShapeDtypeStruct((ETOT,), jnp.float32),
        mesh=_SC_MESH,
        compiler_params=pltpu.CompilerParams(use_tc_tiling_on_sc=False),
        scratch_types=[
            pltpu.VMEM((CH,), jnp.int32),
            pltpu.VMEM((CH,), jnp.int32),
            pltpu.VMEM((CH, 16), jnp.float32),
            pltpu.VMEM((CH, 16), jnp.float32),
            pltpu.VMEM((16,), jnp.float32),
            pltpu.VMEM((CH,), jnp.float32),
            pltpu.VMEM((32,), jnp.float32),
            pltpu.SemaphoreType.DMA,
        ],
    )(xl16, xr16, att16, src, dst)


def _passb2_body(xl_ref, src_ref, dst_ref, w_ref, zd_ref,
                 num_ref, den_ref,
                 si_v, di_v, dri_v, rows_v, dr_v, w_v, num_acc, den_acc, sem):
    c = lax.axis_index("c")
    s = lax.axis_index("s")
    lane = lax.iota(jnp.int32, 16)
    pltpu.sync_copy(zd_ref, num_acc.at[pl.ds(s * NPS, NPS)])

    @pl.when(s == 0)
    def _():
        pltpu.sync_copy(zd_ref, den_acc)

    plsc.subcore_barrier()

    def chunk(i, carry):
        base = c * EPC_B2 + s * (EPC_B2 // NS) + i * CH
        pltpu.sync_copy(src_ref.at[pl.ds(base, CH)], si_v)
        pltpu.sync_copy(dst_ref.at[pl.ds(base, CH)], di_v)
        pltpu.sync_copy(w_ref.at[pl.ds(base, CH)], w_v)

        def adj(t, carry2):
            sl = pl.ds(t * 16, 16)
            dri_v[sl] = lax.shift_right_logical(di_v[sl], 4)
            return carry2
        lax.fori_loop(0, CH // 16, adj, 0)

        pltpu.async_copy(xl_ref.at[si_v], rows_v, sem).wait()

        def grp(t, carry2):
            wg = w_v[pl.ds(t * 16, 16)]
            dmg = jnp.bitwise_and(di_v[pl.ds(t * 16, 16)], 15)
            for j in range(16):
                e = t * 16 + j
                ws = wg[j]
                rows_v[e, :] = rows_v[e, :] * ws
                dr_v[e, :] = jnp.where(lane == dmg[j], ws, 0.0)
            return carry2
        lax.fori_loop(0, CH // 16, grp, 0)

        pltpu.sync_copy(rows_v, num_acc.at[di_v], add=True)
        pltpu.sync_copy(dr_v, den_acc.at[dri_v], add=True)
        return carry
    lax.fori_loop(0, EPC_B2 // NS // CH, chunk, 0)

    plsc.subcore_barrier()
    pltpu.sync_copy(num_acc.at[pl.ds(s * NPS, NPS)],
                    num_ref.at[pl.ds(c * NTOT + s * NPS, NPS)])

    @pl.when(s == 0)
    def _():
        pltpu.sync_copy(den_acc, den_ref.at[pl.ds(c * (NTOT // 16), NTOT // 16)])


def _passb2(xl16, src, dst, w, zd):
    return pl.kernel(
        _passb2_body,
        out_type=[
            jax.ShapeDtypeStruct((2 * NTOT, 16), jnp.float32),
            jax.ShapeDtypeStruct((2 * (NTOT // 16), 16), jnp.float32),
        ],
        mesh=_SC_MESH,
        compiler_params=pltpu.CompilerParams(use_tc_tiling_on_sc=False),
        scratch_types=[
            pltpu.VMEM((CH,), jnp.int32),
            pltpu.VMEM((CH,), jnp.int32),
            pltpu.VMEM((CH,), jnp.int32),
            pltpu.VMEM((CH, 16), jnp.float32),
            pltpu.VMEM((CH, 16), jnp.float32),
            pltpu.VMEM((CH,), jnp.float32),
            pltpu.VMEM_SHARED((NTOT, 16), jnp.float32),
            pltpu.VMEM_SHARED((NTOT // 16, 16), jnp.float32),
            pltpu.SemaphoreType.DMA,
        ],
    )(xl16, src, dst, w, zd)


def _sample_body(np_ref, dp_ref, b2_ref, g_ref, act_ref, sp_ref):
    col = jax.lax.broadcasted_iota(jnp.int32, (16, 128), 1)
    valid = col < NUM_CHOICES
    num = np_ref[0] + np_ref[1]
    den = dp_ref[0] + dp_ref[1]
    logits = num / (den + 1e-16) + b2_ref[...]
    lg = jnp.where(valid, logits, -jnp.inf)
    m = jnp.max(lg, axis=1, keepdims=True)
    ex = jnp.where(valid, jnp.exp(lg - m), 0.0)
    probs = ex / jnp.sum(ex, axis=1, keepdims=True)
    score = jnp.where(valid, jnp.log(probs + 1e-12) + g_ref[...], -jnp.inf)

    actions = []
    for _ in range(NUM_SELECTED):
        mx = jnp.max(score, axis=1, keepdims=True)
        idx = jnp.min(jnp.where(score == mx, col, 128), axis=1, keepdims=True)
        actions.append(idx)
        score = jnp.where(col == idx, -jnp.inf, score)
    act = jnp.concatenate(actions, axis=1)

    row = jax.lax.broadcasted_iota(jnp.int32, (16, 128), 0)
    srow = col
    probs_p = jnp.where(col < NUM_CHOICES, probs, 0.0)
    outs = []
    for j in range(NUM_SELECTED):
        aj = act[:, j : j + 1]
        G = jnp.where(
            ((srow // 8) == (row // 8)) & ((srow % 8) == aj) & (srow < 16), 1.0, 0.0
        )
        Gp = G[:, :16]
        mmj = jnp.dot(Gp, probs_p[:16, :], preferred_element_type=jnp.float32)
        outs.append(mmj[:, j : j + 1])
    sp = jnp.concatenate(outs, axis=1)

    act_ref[...] = jnp.pad(act, ((0, 0), (0, 124)))
    sp_ref[...] = jnp.pad(sp, ((0, 0), (0, 124)))


def _sample(numP, denP, b2p, g16):
    return pl.pallas_call(
        _sample_body,
        out_shape=[
            jax.ShapeDtypeStruct((16, 128), jnp.int32),
            jax.ShapeDtypeStruct((16, 128), jnp.float32),
        ],
    )(numP, denP, b2p, g16)


_SC_MESH = plsc.VectorSubcoreMesh(core_axis_name="c", subcore_axis_name="s")


def _passa_body(xl_ref, xr_ref, att_ref, src_ref, dst_ref, w_ref,
                si_v, di_v, sih_v, dih_v, xll_v, xlh_v, xrl_v, xrh_v,
                att_v, w_v, fold_v, sem):
    c = lax.axis_index("c")
    s = lax.axis_index("s")
    wid = s * NC + c
    pltpu.sync_copy(att_ref, att_v)

    def chunk(i, carry):
        base = wid * EPW_A + i * CH
        pltpu.sync_copy(src_ref.at[pl.ds(base, CH)], si_v)
        pltpu.sync_copy(dst_ref.at[pl.ds(base, CH)], di_v)

        def adj(t, carry2):
            sl = pl.ds(t * 16, 16)
            sih_v[sl] = si_v[sl] + NTOT
            dih_v[sl] = di_v[sl] + NTOT
            return carry2
        lax.fori_loop(0, CH // 16, adj, 0)

        d1 = pltpu.async_copy(xl_ref.at[si_v], xll_v, sem)
        d2 = pltpu.async_copy(xl_ref.at[sih_v], xlh_v, sem)
        d3 = pltpu.async_copy(xr_ref.at[di_v], xrl_v, sem)
        d4 = pltpu.async_copy(xr_ref.at[dih_v], xrh_v, sem)
        d1.wait()
        d2.wait()
        d3.wait()
        d4.wait()

        lane = lax.iota(jnp.int32, 16)
        fold_v[pl.ds(16, 16)] = jnp.zeros((16,), jnp.float32)

        def grp(t, carry2):
            wvec = jnp.zeros((16,), jnp.float32)
            for j in range(16):
                e = t * 16 + j
                acc = jnp.zeros((16,), jnp.float32)
                for k in range(4):
                    sl = pl.ds(k * 16, 16)
                    v = xll_v[e, sl] + xrl_v[e, sl]
                    v = jnp.maximum(v, 0.2 * v)
                    acc = acc + v * att_v[sl]
                    v2 = xlh_v[e, sl] + xrh_v[e, sl]
                    v2 = jnp.maximum(v2, 0.2 * v2)
                    acc = acc + v2 * att_v[pl.ds(64 + k * 16, 16)]
                for sh in (8, 4, 2, 1):
                    fold_v[pl.ds(0, 16)] = acc
                    acc = fold_v[pl.ds(0, 16)] + fold_v[pl.ds(sh, 16)]
                wvec = jnp.where(lane == j, acc[0], wvec)
            w_v[pl.ds(t * 16, 16)] = jnp.exp(wvec)
            return carry2
        lax.fori_loop(0, CH // 16, grp, 0)

        pltpu.sync_copy(w_v, w_ref.at[pl.ds(base, CH)])
        return carry
    lax.fori_loop(0, EPW_A // CH, chunk, 0)


def _passa(xl2, xr2, att, src, dst):
    return pl.kernel(
        _passa_body,
        out_type=jax.ShapeDtypeStruct((ETOT,), jnp.float32),
        mesh=_SC_MESH,
        compiler_params=pltpu.CompilerParams(use_tc_tiling_on_sc=False),
        scratch_types=[
            pltpu.VMEM((CH,), jnp.int32),
            pltpu.VMEM((CH,), jnp.int32),
            pltpu.VMEM((CH,), jnp.int32),
            pltpu.VMEM((CH,), jnp.int32),
            pltpu.VMEM((CH, 64), jnp.float32),
            pltpu.VMEM((CH, 64), jnp.float32),
            pltpu.VMEM((CH, 64), jnp.float32),
            pltpu.VMEM((CH, 64), jnp.float32),
            pltpu.VMEM((128,), jnp.float32),
            pltpu.VMEM((CH,), jnp.float32),
            pltpu.VMEM((32,), jnp.float32),
            pltpu.SemaphoreType.DMA,
        ],
    )(xl2, xr2, att, src, dst)


def _passb_body(xl_ref, src_ref, dst_ref, w_ref, zn_ref, zd_ref,
                num_ref, den_ref,
                si_v, di_v, dri_v, rows_v, dr_v, w_v, num_acc, den_acc, sem):
    c = lax.axis_index("c")
    s = lax.axis_index("s")
    lane = lax.iota(jnp.int32, 16)
    pltpu.sync_copy(zn_ref, num_acc.at[pl.ds(s * NPS, NPS)])

    @pl.when((c == 0) & (s == 0))
    def _():
        pltpu.sync_copy(zd_ref, den_acc)

    plsc.subcore_barrier()

    def chunk(i, carry):
        base = s * EPS_B + i * CH
        pltpu.sync_copy(src_ref.at[pl.ds(base, CH)], si_v)
        pltpu.sync_copy(dst_ref.at[pl.ds(base, CH)], di_v)
        pltpu.sync_copy(w_ref.at[pl.ds(base, CH)], w_v)

        def adj(t, carry2):
            sl = pl.ds(t * 16, 16)
            si_v[sl] = si_v[sl] + c * NTOT
            dri_v[sl] = lax.shift_right_logical(di_v[sl], 4)
            return carry2
        lax.fori_loop(0, CH // 16, adj, 0)

        pltpu.async_copy(xl_ref.at[si_v], rows_v, sem).wait()

        def grp(t, carry2):
            wg = w_v[pl.ds(t * 16, 16)]
            for j in range(16):
                e = t * 16 + j
                ws = wg[j]
                for k in range(4):
                    sl = pl.ds(k * 16, 16)
                    rows_v[e, sl] = rows_v[e, sl] * ws
            return carry2
        lax.fori_loop(0, CH // 16, grp, 0)

        @pl.when(c == 0)
        def _():
            def dgrp(t, carry2):
                wg = w_v[pl.ds(t * 16, 16)]
                dmg = jnp.bitwise_and(di_v[pl.ds(t * 16, 16)], 15)
                for j in range(16):
                    dr_v[t * 16 + j, :] = jnp.where(
                        lane == dmg[j], wg[j], 0.0
                    )
                return carry2
            lax.fori_loop(0, CH // 16, dgrp, 0)

        pltpu.sync_copy(rows_v, num_acc.at[di_v], add=True)

        @pl.when(c == 0)
        def _():
            pltpu.sync_copy(dr_v, den_acc.at[dri_v], add=True)
        return carry
    lax.fori_loop(0, EPS_B // CH, chunk, 0)

    plsc.subcore_barrier()
    pltpu.sync_copy(num_acc.at[pl.ds(s * NPS, NPS)],
                    num_ref.at[pl.ds(c * NTOT + s * NPS, NPS)])

    @pl.when((c == 0) & (s == 0))
    def _():
        pltpu.sync_copy(den_acc, den_ref)


def _passb(xl2, src, dst, w, zn, zd):
    return pl.kernel(
        _passb_body,
        out_type=[
            jax.ShapeDtypeStruct((2 * NTOT, 64), jnp.float32),
            jax.ShapeDtypeStruct((NTOT // 16, 16), jnp.float32),
        ],
        mesh=_SC_MESH,
        compiler_params=pltpu.CompilerParams(use_tc_tiling_on_sc=False),
        scratch_types=[
            pltpu.VMEM((CH,), jnp.int32),
            pltpu.VMEM((CH,), jnp.int32),
            pltpu.VMEM((CH,), jnp.int32),
            pltpu.VMEM((CH, 64), jnp.float32),
            pltpu.VMEM((CH, 16), jnp.float32),
            pltpu.VMEM((CH,), jnp.float32),
            pltpu.VMEM_SHARED((NTOT, 64), jnp.float32),
            pltpu.VMEM_SHARED((NTOT // 16, 16), jnp.float32),
            pltpu.SemaphoreType.DMA,
        ],
    )(xl2, src, dst, w, zn, zd)


def _sc_edge_phase(xl2, xr2, att, src, dst, zn, zd):
    w = _passa(xl2, xr2, att, src, dst)
    num2, den = _passb(xl2, src, dst, w, zn, zd)
    return num2.reshape(2, NTOT, 64), den.reshape(NTOT, 1)


def kernel(x, edge_index, params):
    B, Nn, D = x.shape
    N = B * Nn
    src, dst = edge_index[0], edge_index[1]
    h0 = x.reshape(N, D)
    zn = jnp.zeros((NPS, 64), jnp.float32)
    zd = jnp.zeros((NPS, 16), jnp.float32)

    pa1 = params["a1"]
    xl = _mm_bias_split(h0, pa1["Wl"], pa1["bl"])
    xr = _mm_bias_split(h0, pa1["Wr"], pa1["br"])
    num, den = _sc_edge_phase(xl, xr, pa1["att"][0], src, dst, zn, zd)
    h_a = _combine(num, den, pa1["bias"], 0.01)

    pa2 = params["a2"]
    pad7 = 16 - NUM_CHOICES
    xl2 = _mm_bias16(
        h_a, jnp.pad(pa2["Wl"], ((0, 0), (0, pad7))), jnp.pad(pa2["bl"], (0, pad7))
    )
    xr2 = _mm_bias16(
        h_a, jnp.pad(pa2["Wr"], ((0, 0), (0, pad7))), jnp.pad(pa2["br"], (0, pad7))
    )
    att2 = jnp.pad(pa2["att"][0], (0, pad7))
    w2 = _passa2(xl2, xr2, att2, src, dst)
    num2, den2 = _passb2(xl2, src, dst, w2, zd)

    rows16 = lambda A: jnp.concatenate([A[0:NUM_COLUMNS], A[Nn : Nn + NUM_COLUMNS]], 0)
    numP = jnp.stack([rows16(num2[:NTOT]), rows16(num2[NTOT:])])
    den2f = den2.reshape(2, NTOT)
    denv = jnp.stack([rows16(den2f[0]), rows16(den2f[1])])
    denP = jnp.broadcast_to(denv[:, :, None], (2, 16, 16))
    numP = jnp.pad(numP, ((0, 0), (0, 0), (0, 112)))
    denP = jnp.pad(denP, ((0, 0), (0, 0), (0, 112)))
    b2p = jnp.pad(pa2["bias"], (0, 121)).reshape(1, 128)
    u = jax.random.uniform(
        jax.random.key(123), (B, NUM_COLUMNS, NUM_CHOICES), minval=1e-9, maxval=1.0
    )
    g = -jnp.log(-jnp.log(u))
    g16 = jnp.pad(g.reshape(16, NUM_CHOICES), ((0, 0), (0, 121)))
    act_p, sp_p = _sample(numP, denP, b2p, g16)
    action = act_p[:, :NUM_SELECTED].reshape(B, NUM_COLUMNS, NUM_SELECTED)
    selected_probs = sp_p[:, :NUM_SELECTED].reshape(B, NUM_COLUMNS, NUM_SELECTED)

    pc1 = params["c1"]
    cl = _mm_bias_split(h0, pc1["Wl"], pc1["bl"])
    cr = _mm_bias_split(h0, pc1["Wr"], pc1["br"])
    numc, denc = _sc_edge_phase(cl, cr, pc1["att"][0], src, dst, zn, zd)
    h_c = _combine(numc, denc, pc1["bias"], 0.01)

    pc2 = params["c2"]
    cl2 = _mm_bias_split(h_c, pc2["Wl"], pc2["bl"])
    cr2 = _mm_bias_split(h_c, pc2["Wr"], pc2["br"])
    numc2, denc2 = _sc_edge_phase(cl2, cr2, pc2["att"][0], src, dst, zn, zd)
    vfull = _value_head(numc2, denc2, pc2["bias"], params["fc_W"], params["fc_b"])
    value = vfull[:, :1].reshape(B, Nn, 1)

    return action, selected_probs, value


# speedup vs baseline: 7.4178x; 1.1590x over previous
import functools
import jax
import jax.numpy as jnp
from jax import lax
from jax.experimental import pallas as pl
from jax.experimental.pallas import tpu as pltpu
from jax.experimental.pallas import tpu_sc as plsc

HID = 128
NUM_CHOICES = 7
NUM_SELECTED = 4
NUM_COLUMNS = 8

ROW_BLK = 2000

NTOT = 20000
ETOT = 320000
NC, NS = 2, 16
NW = NC * NS
NPS = NTOT // NS

CH = 400
EPW_A = ETOT // NW
EPS_B = ETOT // NS
EPC_B2 = ETOT // NC


def _mm_bias_body(h_ref, w_ref, b_ref, o_ref):
    o_ref[...] = (
        jnp.dot(h_ref[...], w_ref[...], preferred_element_type=jnp.float32)
        + b_ref[...]
    )


def _mm_split_body(h_ref, w_ref, b_ref, o_ref):
    o_ref[...] = (
        jnp.dot(h_ref[...], w_ref[0], preferred_element_type=jnp.float32)
        + b_ref[0]
    )


def _mm_bias_split(h, W, b):
    N = h.shape[0]
    Ws = jnp.stack([W[:, :64], W[:, 64:]])
    bs = jnp.stack([b[:64], b[64:]]).reshape(2, 1, 64)
    return pl.pallas_call(
        _mm_split_body,
        grid=(N // ROW_BLK, 2),
        in_specs=[
            pl.BlockSpec((ROW_BLK, 128), lambda i, c: (i, 0)),
            pl.BlockSpec((1, 128, 64), lambda i, c: (c, 0, 0)),
            pl.BlockSpec((1, 1, 64), lambda i, c: (c, 0, 0)),
        ],
        out_specs=pl.BlockSpec(
            (ROW_BLK, 64), lambda i, c: (c * (NTOT // ROW_BLK) + i, 0)
        ),
        out_shape=jax.ShapeDtypeStruct((2 * N, 64), jnp.float32),
    )(h, Ws, bs)


def _mm_bias16(h, W16, b16):
    N = h.shape[0]
    return pl.pallas_call(
        _mm_bias_body,
        grid=(N // ROW_BLK,),
        in_specs=[
            pl.BlockSpec((ROW_BLK, 128), lambda i: (i, 0)),
            pl.BlockSpec((128, 16), lambda i: (0, 0)),
            pl.BlockSpec((1, 16), lambda i: (0, 0)),
        ],
        out_specs=pl.BlockSpec((ROW_BLK, 16), lambda i: (i, 0)),
        out_shape=jax.ShapeDtypeStruct((N, 16), jnp.float32),
    )(h, W16, b16.reshape(1, 16))


def _combine_body(slope, num_ref, den_ref, b_ref, o_ref):
    h = jnp.concatenate([num_ref[0], num_ref[1]], axis=1)
    h = h / (den_ref[...] + 1e-16) + b_ref[...]
    if slope is not None:
        h = jnp.maximum(h, slope * h)
    o_ref[...] = h


def _combine(num, den, bias, slope):
    N = num.shape[1]
    return pl.pallas_call(
        functools.partial(_combine_body, slope),
        grid=(N // ROW_BLK,),
        in_specs=[
            pl.BlockSpec((2, ROW_BLK, 64), lambda i: (0, i, 0)),
            pl.BlockSpec((ROW_BLK, 1), lambda i: (i, 0)),
            pl.BlockSpec((1, 128), lambda i: (0, 0)),
        ],
        out_specs=pl.BlockSpec((ROW_BLK, 128), lambda i: (i, 0)),
        out_shape=jax.ShapeDtypeStruct((N, 128), jnp.float32),
    )(num, den, bias.reshape(1, 128))


def _value_body(num_ref, den_ref, b_ref, w_ref, fcb_ref, o_ref):
    h = jnp.concatenate([num_ref[0], num_ref[1]], axis=1)
    h = h / (den_ref[...] + 1e-16) + b_ref[...]
    v = jnp.dot(h, w_ref[...], preferred_element_type=jnp.float32)
    o_ref[...] = v + fcb_ref[0, 0]


def _value_head(num, den, bias, fc_W, fc_b):
    N = num.shape[1]
    fc_pad = jnp.pad(fc_W, ((0, 0), (0, 127)))
    return pl.pallas_call(
        _value_body,
        grid=(N // ROW_BLK,),
        in_specs=[
            pl.BlockSpec((2, ROW_BLK, 64), lambda i: (0, i, 0)),
            pl.BlockSpec((ROW_BLK, 1), lambda i: (i, 0)),
            pl.BlockSpec((1, 128), lambda i: (0, 0)),
            pl.BlockSpec((128, 128), lambda i: (0, 0)),
            pl.BlockSpec((1, 1), lambda i: (0, 0), memory_space=pltpu.SMEM),
        ],
        out_specs=pl.BlockSpec((ROW_BLK, 128), lambda i: (i, 0)),
        out_shape=jax.ShapeDtypeStruct((N, 128), jnp.float32),
    )(num, den, bias.reshape(1, 128), fc_pad, fc_b.reshape(1, 1))


def _passa2_body(xl_ref, xr_ref, att_ref, src_ref, dst_ref, w_ref,
                 si_v, di_v, xls_v, xrs_v, att_v, w_v, fold_v, sem):
    c = lax.axis_index("c")
    s = lax.axis_index("s")
    wid = s * NC + c
    pltpu.sync_copy(att_ref, att_v)

    def chunk(i, carry):
        base = wid * EPW_A + i * CH
        pltpu.sync_copy(src_ref.at[pl.ds(base, CH)], si_v)
        pltpu.sync_copy(dst_ref.at[pl.ds(base, CH)], di_v)
        d1 = pltpu.async_copy(xl_ref.at[si_v], xls_v, sem)
        d2 = pltpu.async_copy(xr_ref.at[di_v], xrs_v, sem)
        d1.wait()
        d2.wait()

        lane = lax.iota(jnp.int32, 16)
        zv = jnp.zeros((16,), jnp.float32)
        for j in range(16):
            fold_v[pl.ds(j * 32 + 16, 16)] = zv

        def grp(t, carry2):
            for j in range(16):
                e = t * 16 + j
                v = xls_v[e, :] + xrs_v[e, :]
                v = jnp.maximum(v, 0.2 * v)
                fold_v[pl.ds(j * 32, 16)] = v * att_v[...]
            for sh in (8, 4, 2, 1):
                for j in range(16):
                    o = j * 32
                    fold_v[pl.ds(o, 16)] = (
                        fold_v[pl.ds(o, 16)] + fold_v[pl.ds(o + sh, 16)]
                    )
            wvec = jnp.zeros((16,), jnp.float32)
            for j in range(16):
                wvec = jnp.where(lane == j, fold_v[pl.ds(j * 32, 16)][0], wvec)
            w_v[pl.ds(t * 16, 16)] = jnp.exp(wvec)
            return carry2
        lax.fori_loop(0, CH // 16, grp, 0)

        pltpu.sync_copy(w_v, w_ref.at[pl.ds(base, CH)])
        return carry
    lax.fori_loop(0, EPW_A // CH, chunk, 0)


def _passa2(xl16, xr16, att16, src, dst):
    return pl.kernel(
        _passa2_body,
        out_type=jax.ShapeDtypeStruct((ETOT,), jnp.float32),
        mesh=_SC_MESH,
        compiler_params=pltpu.CompilerParams(use_tc_tiling_on_sc=False),
        scratch_types=[
            pltpu.VMEM((CH,), jnp.int32),
            pltpu.VMEM((CH,), jnp.int32),
            pltpu.VMEM((CH, 16), jnp.float32),
            pltpu.VMEM((CH, 16), jnp.float32),
            pltpu.VMEM((16,), jnp.float32),
            pltpu.VMEM((CH,), jnp.float32),
            pltpu.VMEM((512,), jnp.float32),
            pltpu.SemaphoreType.DMA,
        ],
    )(xl16, xr16, att16, src, dst)


def _passb2_body(xl_ref, src_ref, dst_ref, w_ref, zd_ref,
                 num_ref, den_ref,
                 si_v, di_v, dri_v, rows_v, dr_v, w_v, num_acc, den_acc, sem):
    c = lax.axis_index("c")
    s = lax.axis_index("s")
    lane = lax.iota(jnp.int32, 16)
    pltpu.sync_copy(zd_ref, num_acc.at[pl.ds(s * NPS, NPS)])

    @pl.when(s == 0)
    def _():
        pltpu.sync_copy(zd_ref, den_acc)

    plsc.subcore_barrier()

    def chunk(i, carry):
        base = c * EPC_B2 + s * (EPC_B2 // NS) + i * CH
        pltpu.sync_copy(src_ref.at[pl.ds(base, CH)], si_v)
        pltpu.sync_copy(dst_ref.at[pl.ds(base, CH)], di_v)
        pltpu.sync_copy(w_ref.at[pl.ds(base, CH)], w_v)

        def adj(t, carry2):
            sl = pl.ds(t * 16, 16)
            dri_v[sl] = lax.shift_right_logical(di_v[sl], 4)
            return carry2
        lax.fori_loop(0, CH // 16, adj, 0)

        pltpu.async_copy(xl_ref.at[si_v], rows_v, sem).wait()

        def grp(t, carry2):
            wg = w_v[pl.ds(t * 16, 16)]
            dmg = jnp.bitwise_and(di_v[pl.ds(t * 16, 16)], 15)
            for j in range(16):
                e = t * 16 + j
                ws = wg[j]
                rows_v[e, :] = rows_v[e, :] * ws
                dr_v[e, :] = jnp.where(lane == dmg[j], ws, 0.0)
            return carry2
        lax.fori_loop(0, CH // 16, grp, 0)

        pltpu.sync_copy(rows_v, num_acc.at[di_v], add=True)
        pltpu.sync_copy(dr_v, den_acc.at[dri_v], add=True)
        return carry
    lax.fori_loop(0, EPC_B2 // NS // CH, chunk, 0)

    plsc.subcore_barrier()
    pltpu.sync_copy(num_acc.at[pl.ds(s * NPS, NPS)],
                    num_ref.at[pl.ds(c * NTOT + s * NPS, NPS)])

    @pl.when(s == 0)
    def _():
        pltpu.sync_copy(den_acc, den_ref.at[pl.ds(c * (NTOT // 16), NTOT // 16)])


def _passb2(xl16, src, dst, w, zd):
    return pl.kernel(
        _passb2_body,
        out_type=[
            jax.ShapeDtypeStruct((2 * NTOT, 16), jnp.float32),
            jax.ShapeDtypeStruct((2 * (NTOT // 16), 16), jnp.float32),
        ],
        mesh=_SC_MESH,
        compiler_params=pltpu.CompilerParams(use_tc_tiling_on_sc=False),
        scratch_types=[
            pltpu.VMEM((CH,), jnp.int32),
            pltpu.VMEM((CH,), jnp.int32),
            pltpu.VMEM((CH,), jnp.int32),
            pltpu.VMEM((CH, 16), jnp.float32),
            pltpu.VMEM((CH, 16), jnp.float32),
            pltpu.VMEM((CH,), jnp.float32),
            pltpu.VMEM_SHARED((NTOT, 16), jnp.float32),
            pltpu.VMEM_SHARED((NTOT // 16, 16), jnp.float32),
            pltpu.SemaphoreType.DMA,
        ],
    )(xl16, src, dst, w, zd)


def _sample_body(np_ref, dp_ref, b2_ref, g_ref, act_ref, sp_ref):
    col = jax.lax.broadcasted_iota(jnp.int32, (16, 128), 1)
    valid = col < NUM_CHOICES
    num = np_ref[0] + np_ref[1]
    den = dp_ref[0] + dp_ref[1]
    logits = num / (den + 1e-16) + b2_ref[...]
    lg = jnp.where(valid, logits, -jnp.inf)
    m = jnp.max(lg, axis=1, keepdims=True)
    ex = jnp.where(valid, jnp.exp(lg - m), 0.0)
    probs = ex / jnp.sum(ex, axis=1, keepdims=True)
    score = jnp.where(valid, jnp.log(probs + 1e-12) + g_ref[...], -jnp.inf)

    actions = []
    for _ in range(NUM_SELECTED):
        mx = jnp.max(score, axis=1, keepdims=True)
        idx = jnp.min(jnp.where(score == mx, col, 128), axis=1, keepdims=True)
        actions.append(idx)
        score = jnp.where(col == idx, -jnp.inf, score)
    act = jnp.concatenate(actions, axis=1)

    row = jax.lax.broadcasted_iota(jnp.int32, (16, 128), 0)
    srow = col
    probs_p = jnp.where(col < NUM_CHOICES, probs, 0.0)
    outs = []
    for j in range(NUM_SELECTED):
        aj = act[:, j : j + 1]
        G = jnp.where(
            ((srow // 8) == (row // 8)) & ((srow % 8) == aj) & (srow < 16), 1.0, 0.0
        )
        Gp = G[:, :16]
        mmj = jnp.dot(Gp, probs_p[:16, :], preferred_element_type=jnp.float32)
        outs.append(mmj[:, j : j + 1])
    sp = jnp.concatenate(outs, axis=1)

    act_ref[...] = jnp.pad(act, ((0, 0), (0, 124)))
    sp_ref[...] = jnp.pad(sp, ((0, 0), (0, 124)))


def _sample(numP, denP, b2p, g16):
    return pl.pallas_call(
        _sample_body,
        out_shape=[
            jax.ShapeDtypeStruct((16, 128), jnp.int32),
            jax.ShapeDtypeStruct((16, 128), jnp.float32),
        ],
    )(numP, denP, b2p, g16)


_SC_MESH = plsc.VectorSubcoreMesh(core_axis_name="c", subcore_axis_name="s")


def _passa_body(xl_ref, xr_ref, att_ref, src_ref, dst_ref, w_ref,
                si_v, di_v, sih_v, dih_v, xll_v, xlh_v, xrl_v, xrh_v,
                att_v, w_v, fold_v, sem):
    c = lax.axis_index("c")
    s = lax.axis_index("s")
    wid = s * NC + c
    pltpu.sync_copy(att_ref, att_v)

    def chunk(i, carry):
        base = wid * EPW_A + i * CH
        pltpu.sync_copy(src_ref.at[pl.ds(base, CH)], si_v)
        pltpu.sync_copy(dst_ref.at[pl.ds(base, CH)], di_v)

        def adj(t, carry2):
            sl = pl.ds(t * 16, 16)
            sih_v[sl] = si_v[sl] + NTOT
            dih_v[sl] = di_v[sl] + NTOT
            return carry2
        lax.fori_loop(0, CH // 16, adj, 0)

        d1 = pltpu.async_copy(xl_ref.at[si_v], xll_v, sem)
        d2 = pltpu.async_copy(xl_ref.at[sih_v], xlh_v, sem)
        d3 = pltpu.async_copy(xr_ref.at[di_v], xrl_v, sem)
        d4 = pltpu.async_copy(xr_ref.at[dih_v], xrh_v, sem)
        d1.wait()
        d2.wait()
        d3.wait()
        d4.wait()

        lane = lax.iota(jnp.int32, 16)
        zv = jnp.zeros((16,), jnp.float32)
        for j in range(16):
            fold_v[pl.ds(j * 32 + 16, 16)] = zv

        def grp(t, carry2):
            for j in range(16):
                e = t * 16 + j
                acc = jnp.zeros((16,), jnp.float32)
                for k in range(4):
                    sl = pl.ds(k * 16, 16)
                    v = xll_v[e, sl] + xrl_v[e, sl]
                    v = jnp.maximum(v, 0.2 * v)
                    acc = acc + v * att_v[sl]
                    v2 = xlh_v[e, sl] + xrh_v[e, sl]
                    v2 = jnp.maximum(v2, 0.2 * v2)
                    acc = acc + v2 * att_v[pl.ds(64 + k * 16, 16)]
                fold_v[pl.ds(j * 32, 16)] = acc
            for sh in (8, 4, 2, 1):
                for j in range(16):
                    o = j * 32
                    fold_v[pl.ds(o, 16)] = (
                        fold_v[pl.ds(o, 16)] + fold_v[pl.ds(o + sh, 16)]
                    )
            wvec = jnp.zeros((16,), jnp.float32)
            for j in range(16):
                wvec = jnp.where(lane == j, fold_v[pl.ds(j * 32, 16)][0], wvec)
            w_v[pl.ds(t * 16, 16)] = jnp.exp(wvec)
            return carry2
        lax.fori_loop(0, CH // 16, grp, 0)

        pltpu.sync_copy(w_v, w_ref.at[pl.ds(base, CH)])
        return carry
    lax.fori_loop(0, EPW_A // CH, chunk, 0)


def _passa(xl2, xr2, att, src, dst):
    return pl.kernel(
        _passa_body,
        out_type=jax.ShapeDtypeStruct((ETOT,), jnp.float32),
        mesh=_SC_MESH,
        compiler_params=pltpu.CompilerParams(use_tc_tiling_on_sc=False),
        scratch_types=[
            pltpu.VMEM((CH,), jnp.int32),
            pltpu.VMEM((CH,), jnp.int32),
            pltpu.VMEM((CH,), jnp.int32),
            pltpu.VMEM((CH,), jnp.int32),
            pltpu.VMEM((CH, 64), jnp.float32),
            pltpu.VMEM((CH, 64), jnp.float32),
            pltpu.VMEM((CH, 64), jnp.float32),
            pltpu.VMEM((CH, 64), jnp.float32),
            pltpu.VMEM((128,), jnp.float32),
            pltpu.VMEM((CH,), jnp.float32),
            pltpu.VMEM((512,), jnp.float32),
            pltpu.SemaphoreType.DMA,
        ],
    )(xl2, xr2, att, src, dst)


def _passb_body(xl_ref, src_ref, dst_ref, w_ref, zn_ref, zd_ref,
                num_ref, den_ref,
                si_v, di_v, dri_v, rows_v, dr_v, w_v, num_acc, den_acc, sem):
    c = lax.axis_index("c")
    s = lax.axis_index("s")
    lane = lax.iota(jnp.int32, 16)
    pltpu.sync_copy(zn_ref, num_acc.at[pl.ds(s * NPS, NPS)])

    @pl.when((c == 0) & (s == 0))
    def _():
        pltpu.sync_copy(zd_ref, den_acc)

    plsc.subcore_barrier()

    def chunk(i, carry):
        base = s * EPS_B + i * CH
        pltpu.sync_copy(src_ref.at[pl.ds(base, CH)], si_v)
        pltpu.sync_copy(dst_ref.at[pl.ds(base, CH)], di_v)
        pltpu.sync_copy(w_ref.at[pl.ds(base, CH)], w_v)

        def adj(t, carry2):
            sl = pl.ds(t * 16, 16)
            si_v[sl] = si_v[sl] + c * NTOT
            dri_v[sl] = lax.shift_right_logical(di_v[sl], 4)
            return carry2
        lax.fori_loop(0, CH // 16, adj, 0)

        pltpu.async_copy(xl_ref.at[si_v], rows_v, sem).wait()

        def grp(t, carry2):
            wg = w_v[pl.ds(t * 16, 16)]
            for j in range(16):
                e = t * 16 + j
                ws = wg[j]
                for k in range(4):
                    sl = pl.ds(k * 16, 16)
                    rows_v[e, sl] = rows_v[e, sl] * ws
            return carry2
        lax.fori_loop(0, CH // 16, grp, 0)

        @pl.when(c == 0)
        def _():
            def dgrp(t, carry2):
                wg = w_v[pl.ds(t * 16, 16)]
                dmg = jnp.bitwise_and(di_v[pl.ds(t * 16, 16)], 15)
                for j in range(16):
                    dr_v[t * 16 + j, :] = jnp.where(
                        lane == dmg[j], wg[j], 0.0
                    )
                return carry2
            lax.fori_loop(0, CH // 16, dgrp, 0)

        pltpu.sync_copy(rows_v, num_acc.at[di_v], add=True)

        @pl.when(c == 0)
        def _():
            pltpu.sync_copy(dr_v, den_acc.at[dri_v], add=True)
        return carry
    lax.fori_loop(0, EPS_B // CH, chunk, 0)

    plsc.subcore_barrier()
    pltpu.sync_copy(num_acc.at[pl.ds(s * NPS, NPS)],
                    num_ref.at[pl.ds(c * NTOT + s * NPS, NPS)])

    @pl.when((c == 0) & (s == 0))
    def _():
        pltpu.sync_copy(den_acc, den_ref)


def _passb(xl2, src, dst, w, zn, zd):
    return pl.kernel(
        _passb_body,
        out_type=[
            jax.ShapeDtypeStruct((2 * NTOT, 64), jnp.float32),
            jax.ShapeDtypeStruct((NTOT // 16, 16), jnp.float32),
        ],
        mesh=_SC_MESH,
        compiler_params=pltpu.CompilerParams(use_tc_tiling_on_sc=False),
        scratch_types=[
            pltpu.VMEM((CH,), jnp.int32),
            pltpu.VMEM((CH,), jnp.int32),
            pltpu.VMEM((CH,), jnp.int32),
            pltpu.VMEM((CH, 64), jnp.float32),
            pltpu.VMEM((CH, 16), jnp.float32),
            pltpu.VMEM((CH,), jnp.float32),
            pltpu.VMEM_SHARED((NTOT, 64), jnp.float32),
            pltpu.VMEM_SHARED((NTOT // 16, 16), jnp.float32),
            pltpu.SemaphoreType.DMA,
        ],
    )(xl2, src, dst, w, zn, zd)


def _sc_edge_phase(xl2, xr2, att, src, dst, zn, zd):
    w = _passa(xl2, xr2, att, src, dst)
    num2, den = _passb(xl2, src, dst, w, zn, zd)
    return num2.reshape(2, NTOT, 64), den.reshape(NTOT, 1)


def kernel(x, edge_index, params):
    B, Nn, D = x.shape
    N = B * Nn
    src, dst = edge_index[0], edge_index[1]
    h0 = x.reshape(N, D)
    zn = jnp.zeros((NPS, 64), jnp.float32)
    zd = jnp.zeros((NPS, 16), jnp.float32)

    pa1 = params["a1"]
    xl = _mm_bias_split(h0, pa1["Wl"], pa1["bl"])
    xr = _mm_bias_split(h0, pa1["Wr"], pa1["br"])
    num, den = _sc_edge_phase(xl, xr, pa1["att"][0], src, dst, zn, zd)
    h_a = _combine(num, den, pa1["bias"], 0.01)

    pa2 = params["a2"]
    pad7 = 16 - NUM_CHOICES
    xl2 = _mm_bias16(
        h_a, jnp.pad(pa2["Wl"], ((0, 0), (0, pad7))), jnp.pad(pa2["bl"], (0, pad7))
    )
    xr2 = _mm_bias16(
        h_a, jnp.pad(pa2["Wr"], ((0, 0), (0, pad7))), jnp.pad(pa2["br"], (0, pad7))
    )
    att2 = jnp.pad(pa2["att"][0], (0, pad7))
    w2 = _passa2(xl2, xr2, att2, src, dst)
    num2, den2 = _passb2(xl2, src, dst, w2, zd)

    rows16 = lambda A: jnp.concatenate([A[0:NUM_COLUMNS], A[Nn : Nn + NUM_COLUMNS]], 0)
    numP = jnp.stack([rows16(num2[:NTOT]), rows16(num2[NTOT:])])
    den2f = den2.reshape(2, NTOT)
    denv = jnp.stack([rows16(den2f[0]), rows16(den2f[1])])
    denP = jnp.broadcast_to(denv[:, :, None], (2, 16, 16))
    numP = jnp.pad(numP, ((0, 0), (0, 0), (0, 112)))
    denP = jnp.pad(denP, ((0, 0), (0, 0), (0, 112)))
    b2p = jnp.pad(pa2["bias"], (0, 121)).reshape(1, 128)
    u = jax.random.uniform(
        jax.random.key(123), (B, NUM_COLUMNS, NUM_CHOICES), minval=1e-9, maxval=1.0
    )
    g = -jnp.log(-jnp.log(u))
    g16 = jnp.pad(g.reshape(16, NUM_CHOICES), ((0, 0), (0, 121)))
    act_p, sp_p = _sample(numP, denP, b2p, g16)
    action = act_p[:, :NUM_SELECTED].reshape(B, NUM_COLUMNS, NUM_SELECTED)
    selected_probs = sp_p[:, :NUM_SELECTED].reshape(B, NUM_COLUMNS, NUM_SELECTED)

    pc1 = params["c1"]
    cl = _mm_bias_split(h0, pc1["Wl"], pc1["bl"])
    cr = _mm_bias_split(h0, pc1["Wr"], pc1["br"])
    numc, denc = _sc_edge_phase(cl, cr, pc1["att"][0], src, dst, zn, zd)
    h_c = _combine(numc, denc, pc1["bias"], 0.01)

    pc2 = params["c2"]
    cl2 = _mm_bias_split(h_c, pc2["Wl"], pc2["bl"])
    cr2 = _mm_bias_split(h_c, pc2["Wr"], pc2["br"])
    numc2, denc2 = _sc_edge_phase(cl2, cr2, pc2["att"][0], src, dst, zn, zd)
    vfull = _value_head(numc2, denc2, pc2["bias"], params["fc_W"], params["fc_b"])
    value = vfull[:, :1].reshape(B, Nn, 1)

    return action, selected_probs, value


# speedup vs baseline: 7.4250x; 1.0010x over previous
import functools
import jax
import jax.numpy as jnp
from jax import lax
from jax.experimental import pallas as pl
from jax.experimental.pallas import tpu as pltpu
from jax.experimental.pallas import tpu_sc as plsc

HID = 128
NUM_CHOICES = 7
NUM_SELECTED = 4
NUM_COLUMNS = 8

ROW_BLK = 2000

NTOT = 20000
ETOT = 320000
NC, NS = 2, 16
NW = NC * NS
NPS = NTOT // NS

CH = 400
EPW_A = ETOT // NW
EPS_B = ETOT // NS
EPC_B2 = ETOT // NC
CHA = 200


def _mm_bias_body(h_ref, w_ref, b_ref, o_ref):
    o_ref[...] = (
        jnp.dot(h_ref[...], w_ref[...], preferred_element_type=jnp.float32)
        + b_ref[...]
    )


def _mm_split_body(h_ref, w_ref, b_ref, o_ref):
    o_ref[...] = (
        jnp.dot(h_ref[...], w_ref[0], preferred_element_type=jnp.float32)
        + b_ref[0]
    )


def _mm_bias_split(h, W, b):
    N = h.shape[0]
    Ws = jnp.stack([W[:, :64], W[:, 64:]])
    bs = jnp.stack([b[:64], b[64:]]).reshape(2, 1, 64)
    return pl.pallas_call(
        _mm_split_body,
        grid=(N // ROW_BLK, 2),
        in_specs=[
            pl.BlockSpec((ROW_BLK, 128), lambda i, c: (i, 0)),
            pl.BlockSpec((1, 128, 64), lambda i, c: (c, 0, 0)),
            pl.BlockSpec((1, 1, 64), lambda i, c: (c, 0, 0)),
        ],
        out_specs=pl.BlockSpec(
            (ROW_BLK, 64), lambda i, c: (c * (NTOT // ROW_BLK) + i, 0)
        ),
        out_shape=jax.ShapeDtypeStruct((2 * N, 64), jnp.float32),
    )(h, Ws, bs)


def _mm_bias16(h, W16, b16):
    N = h.shape[0]
    return pl.pallas_call(
        _mm_bias_body,
        grid=(N // ROW_BLK,),
        in_specs=[
            pl.BlockSpec((ROW_BLK, 128), lambda i: (i, 0)),
            pl.BlockSpec((128, 16), lambda i: (0, 0)),
            pl.BlockSpec((1, 16), lambda i: (0, 0)),
        ],
        out_specs=pl.BlockSpec((ROW_BLK, 16), lambda i: (i, 0)),
        out_shape=jax.ShapeDtypeStruct((N, 16), jnp.float32),
    )(h, W16, b16.reshape(1, 16))


def _combine_body(slope, num_ref, den_ref, b_ref, o_ref):
    h = jnp.concatenate([num_ref[0], num_ref[1]], axis=1)
    h = h / (den_ref[...] + 1e-16) + b_ref[...]
    if slope is not None:
        h = jnp.maximum(h, slope * h)
    o_ref[...] = h


def _combine(num, den, bias, slope):
    N = num.shape[1]
    return pl.pallas_call(
        functools.partial(_combine_body, slope),
        grid=(N // ROW_BLK,),
        in_specs=[
            pl.BlockSpec((2, ROW_BLK, 64), lambda i: (0, i, 0)),
            pl.BlockSpec((ROW_BLK, 1), lambda i: (i, 0)),
            pl.BlockSpec((1, 128), lambda i: (0, 0)),
        ],
        out_specs=pl.BlockSpec((ROW_BLK, 128), lambda i: (i, 0)),
        out_shape=jax.ShapeDtypeStruct((N, 128), jnp.float32),
    )(num, den, bias.reshape(1, 128))


def _value_body(num_ref, den_ref, b_ref, w_ref, fcb_ref, o_ref):
    h = jnp.concatenate([num_ref[0], num_ref[1]], axis=1)
    h = h / (den_ref[...] + 1e-16) + b_ref[...]
    v = jnp.dot(h, w_ref[...], preferred_element_type=jnp.float32)
    o_ref[...] = v + fcb_ref[0, 0]


def _value_head(num, den, bias, fc_W, fc_b):
    N = num.shape[1]
    fc_pad = jnp.pad(fc_W, ((0, 0), (0, 127)))
    return pl.pallas_call(
        _value_body,
        grid=(N // ROW_BLK,),
        in_specs=[
            pl.BlockSpec((2, ROW_BLK, 64), lambda i: (0, i, 0)),
            pl.BlockSpec((ROW_BLK, 1), lambda i: (i, 0)),
            pl.BlockSpec((1, 128), lambda i: (0, 0)),
            pl.BlockSpec((128, 128), lambda i: (0, 0)),
            pl.BlockSpec((1, 1), lambda i: (0, 0), memory_space=pltpu.SMEM),
        ],
        out_specs=pl.BlockSpec((ROW_BLK, 128), lambda i: (i, 0)),
        out_shape=jax.ShapeDtypeStruct((N, 128), jnp.float32),
    )(num, den, bias.reshape(1, 128), fc_pad, fc_b.reshape(1, 1))


def _passa2_body(xl_ref, xr_ref, att_ref, src_ref, dst_ref, w_ref,
                 si_v, di_v, xls_v, xrs_v, att_v, w_v, fold_v, sem):
    c = lax.axis_index("c")
    s = lax.axis_index("s")
    wid = s * NC + c
    pltpu.sync_copy(att_ref, att_v)

    def chunk(i, carry):
        base = wid * EPW_A + i * CH
        pltpu.sync_copy(src_ref.at[pl.ds(base, CH)], si_v)
        pltpu.sync_copy(dst_ref.at[pl.ds(base, CH)], di_v)
        d1 = pltpu.async_copy(xl_ref.at[si_v], xls_v, sem)
        d2 = pltpu.async_copy(xr_ref.at[di_v], xrs_v, sem)
        d1.wait()
        d2.wait()

        lane = lax.iota(jnp.int32, 16)
        zv = jnp.zeros((16,), jnp.float32)
        for j in range(16):
            fold_v[pl.ds(j * 32 + 16, 16)] = zv

        def grp(t, carry2):
            for j in range(16):
                e = t * 16 + j
                v = xls_v[e, :] + xrs_v[e, :]
                v = jnp.maximum(v, 0.2 * v)
                fold_v[pl.ds(j * 32, 16)] = v * att_v[...]
            for sh in (8, 4, 2, 1):
                for j in range(16):
                    o = j * 32
                    fold_v[pl.ds(o, 16)] = (
                        fold_v[pl.ds(o, 16)] + fold_v[pl.ds(o + sh, 16)]
                    )
            wvec = jnp.zeros((16,), jnp.float32)
            for j in range(16):
                wvec = jnp.where(lane == j, fold_v[pl.ds(j * 32, 16)][0], wvec)
            w_v[pl.ds(t * 16, 16)] = jnp.exp(wvec)
            return carry2
        lax.fori_loop(0, CH // 16, grp, 0)

        pltpu.sync_copy(w_v, w_ref.at[pl.ds(base, CH)])
        return carry
    lax.fori_loop(0, EPW_A // CH, chunk, 0)


def _passa2(xl16, xr16, att16, src, dst):
    return pl.kernel(
        _passa2_body,
        out_type=jax.ShapeDtypeStruct((ETOT,), jnp.float32),
        mesh=_SC_MESH,
        compiler_params=pltpu.CompilerParams(use_tc_tiling_on_sc=False),
        scratch_types=[
            pltpu.VMEM((CH,), jnp.int32),
            pltpu.VMEM((CH,), jnp.int32),
            pltpu.VMEM((CH, 16), jnp.float32),
            pltpu.VMEM((CH, 16), jnp.float32),
            pltpu.VMEM((16,), jnp.float32),
            pltpu.VMEM((CH,), jnp.float32),
            pltpu.VMEM((512,), jnp.float32),
            pltpu.SemaphoreType.DMA,
        ],
    )(xl16, xr16, att16, src, dst)


def _passb2_body(xl_ref, src_ref, dst_ref, w_ref, zd_ref,
                 num_ref, den_ref,
                 si_v, di_v, dri_v, rows_v, dr_v, w_v, num_acc, den_acc, sem):
    c = lax.axis_index("c")
    s = lax.axis_index("s")
    lane = lax.iota(jnp.int32, 16)
    pltpu.sync_copy(zd_ref, num_acc.at[pl.ds(s * NPS, NPS)])

    @pl.when(s == 0)
    def _():
        pltpu.sync_copy(zd_ref, den_acc)

    plsc.subcore_barrier()

    def chunk(i, carry):
        base = c * EPC_B2 + s * (EPC_B2 // NS) + i * CH
        pltpu.sync_copy(src_ref.at[pl.ds(base, CH)], si_v)
        pltpu.sync_copy(dst_ref.at[pl.ds(base, CH)], di_v)
        pltpu.sync_copy(w_ref.at[pl.ds(base, CH)], w_v)

        def adj(t, carry2):
            sl = pl.ds(t * 16, 16)
            dri_v[sl] = lax.shift_right_logical(di_v[sl], 4)
            return carry2
        lax.fori_loop(0, CH // 16, adj, 0)

        pltpu.async_copy(xl_ref.at[si_v], rows_v, sem).wait()

        def grp(t, carry2):
            wg = w_v[pl.ds(t * 16, 16)]
            dmg = jnp.bitwise_and(di_v[pl.ds(t * 16, 16)], 15)
            for j in range(16):
                e = t * 16 + j
                ws = wg[j]
                rows_v[e, :] = rows_v[e, :] * ws
                dr_v[e, :] = jnp.where(lane == dmg[j], ws, 0.0)
            return carry2
        lax.fori_loop(0, CH // 16, grp, 0)

        pltpu.sync_copy(rows_v, num_acc.at[di_v], add=True)
        pltpu.sync_copy(dr_v, den_acc.at[dri_v], add=True)
        return carry
    lax.fori_loop(0, EPC_B2 // NS // CH, chunk, 0)

    plsc.subcore_barrier()
    pltpu.sync_copy(num_acc.at[pl.ds(s * NPS, NPS)],
                    num_ref.at[pl.ds(c * NTOT + s * NPS, NPS)])

    @pl.when(s == 0)
    def _():
        pltpu.sync_copy(den_acc, den_ref.at[pl.ds(c * (NTOT // 16), NTOT // 16)])


def _passb2(xl16, src, dst, w, zd):
    return pl.kernel(
        _passb2_body,
        out_type=[
            jax.ShapeDtypeStruct((2 * NTOT, 16), jnp.float32),
            jax.ShapeDtypeStruct((2 * (NTOT // 16), 16), jnp.float32),
        ],
        mesh=_SC_MESH,
        compiler_params=pltpu.CompilerParams(use_tc_tiling_on_sc=False),
        scratch_types=[
            pltpu.VMEM((CH,), jnp.int32),
            pltpu.VMEM((CH,), jnp.int32),
            pltpu.VMEM((CH,), jnp.int32),
            pltpu.VMEM((CH, 16), jnp.float32),
            pltpu.VMEM((CH, 16), jnp.float32),
            pltpu.VMEM((CH,), jnp.float32),
            pltpu.VMEM_SHARED((NTOT, 16), jnp.float32),
            pltpu.VMEM_SHARED((NTOT // 16, 16), jnp.float32),
            pltpu.SemaphoreType.DMA,
        ],
    )(xl16, src, dst, w, zd)


def _sample_body(np_ref, dp_ref, b2_ref, g_ref, act_ref, sp_ref):
    col = jax.lax.broadcasted_iota(jnp.int32, (16, 128), 1)
    valid = col < NUM_CHOICES
    num = np_ref[0] + np_ref[1]
    den = dp_ref[0] + dp_ref[1]
    logits = num / (den + 1e-16) + b2_ref[...]
    lg = jnp.where(valid, logits, -jnp.inf)
    m = jnp.max(lg, axis=1, keepdims=True)
    ex = jnp.where(valid, jnp.exp(lg - m), 0.0)
    probs = ex / jnp.sum(ex, axis=1, keepdims=True)
    score = jnp.where(valid, jnp.log(probs + 1e-12) + g_ref[...], -jnp.inf)

    actions = []
    for _ in range(NUM_SELECTED):
        mx = jnp.max(score, axis=1, keepdims=True)
        idx = jnp.min(jnp.where(score == mx, col, 128), axis=1, keepdims=True)
        actions.append(idx)
        score = jnp.where(col == idx, -jnp.inf, score)
    act = jnp.concatenate(actions, axis=1)

    row = jax.lax.broadcasted_iota(jnp.int32, (16, 128), 0)
    srow = col
    probs_p = jnp.where(col < NUM_CHOICES, probs, 0.0)
    outs = []
    for j in range(NUM_SELECTED):
        aj = act[:, j : j + 1]
        G = jnp.where(
            ((srow // 8) == (row // 8)) & ((srow % 8) == aj) & (srow < 16), 1.0, 0.0
        )
        Gp = G[:, :16]
        mmj = jnp.dot(Gp, probs_p[:16, :], preferred_element_type=jnp.float32)
        outs.append(mmj[:, j : j + 1])
    sp = jnp.concatenate(outs, axis=1)

    act_ref[...] = jnp.pad(act, ((0, 0), (0, 124)))
    sp_ref[...] = jnp.pad(sp, ((0, 0), (0, 124)))


def _sample(numP, denP, b2p, g16):
    return pl.pallas_call(
        _sample_body,
        out_shape=[
            jax.ShapeDtypeStruct((16, 128), jnp.int32),
            jax.ShapeDtypeStruct((16, 128), jnp.float32),
        ],
    )(numP, denP, b2p, g16)


_SC_MESH = plsc.VectorSubcoreMesh(core_axis_name="c", subcore_axis_name="s")


def _passa_body(xl_ref, xr_ref, att_ref, src_ref, dst_ref, w_ref,
                si_v, di_v, sih_v, dih_v, xll_v, xlh_v, xrl_v, xrh_v,
                att_v, w_v, fold_v, sem):
    c = lax.axis_index("c")
    s = lax.axis_index("s")
    wid = s * NC + c
    pltpu.sync_copy(att_ref, att_v)
    lane = lax.iota(jnp.int32, 16)
    zv = jnp.zeros((16,), jnp.float32)
    for j in range(16):
        fold_v[pl.ds(j * 32 + 16, 16)] = zv

    def chunk(i, carry):
        base = wid * EPW_A + i * CH
        pltpu.sync_copy(src_ref.at[pl.ds(base, CH)], si_v)
        pltpu.sync_copy(dst_ref.at[pl.ds(base, CH)], di_v)

        def adj(t, carry2):
            sl = pl.ds(t * 16, 16)
            sih_v[sl] = si_v[sl] + NTOT
            dih_v[sl] = di_v[sl] + NTOT
            return carry2
        lax.fori_loop(0, CH // 16, adj, 0)

        d1 = pltpu.async_copy(xl_ref.at[si_v], xll_v, sem)
        d2 = pltpu.async_copy(xl_ref.at[sih_v], xlh_v, sem)
        d3 = pltpu.async_copy(xr_ref.at[di_v], xrl_v, sem)
        d4 = pltpu.async_copy(xr_ref.at[dih_v], xrh_v, sem)
        d1.wait()
        d2.wait()
        d3.wait()
        d4.wait()

        def grp(t, carry2):
            for j in range(16):
                e = t * 16 + j
                acc = jnp.zeros((16,), jnp.float32)
                for k in range(4):
                    sl = pl.ds(k * 16, 16)
                    v = xll_v[e, sl] + xrl_v[e, sl]
                    v = jnp.maximum(v, 0.2 * v)
                    acc = acc + v * att_v[sl]
                    v2 = xlh_v[e, sl] + xrh_v[e, sl]
                    v2 = jnp.maximum(v2, 0.2 * v2)
                    acc = acc + v2 * att_v[pl.ds(64 + k * 16, 16)]
                fold_v[pl.ds(j * 32, 16)] = acc
            for sh in (8, 4, 2, 1):
                for j in range(16):
                    o = j * 32
                    fold_v[pl.ds(o, 16)] = (
                        fold_v[pl.ds(o, 16)] + fold_v[pl.ds(o + sh, 16)]
                    )
            wvec = jnp.zeros((16,), jnp.float32)
            for j in range(16):
                wvec = jnp.where(lane == j, fold_v[pl.ds(j * 32, 16)][0], wvec)
            w_v[pl.ds(t * 16, 16)] = jnp.exp(wvec)
            return carry2
        lax.fori_loop(0, CH // 16, grp, 0)

        pltpu.sync_copy(w_v, w_ref.at[pl.ds(base, CH)])
        return carry
    lax.fori_loop(0, EPW_A // CH, chunk, 0)


def _passa(xl2, xr2, att, src, dst):
    return pl.kernel(
        _passa_body,
        out_type=jax.ShapeDtypeStruct((ETOT,), jnp.float32),
        mesh=_SC_MESH,
        compiler_params=pltpu.CompilerParams(use_tc_tiling_on_sc=False),
        scratch_types=[
            pltpu.VMEM((CH,), jnp.int32),
            pltpu.VMEM((CH,), jnp.int32),
            pltpu.VMEM((CH,), jnp.int32),
            pltpu.VMEM((CH,), jnp.int32),
            pltpu.VMEM((CH, 64), jnp.float32),
            pltpu.VMEM((CH, 64), jnp.float32),
            pltpu.VMEM((CH, 64), jnp.float32),
            pltpu.VMEM((CH, 64), jnp.float32),
            pltpu.VMEM((128,), jnp.float32),
            pltpu.VMEM((CH,), jnp.float32),
            pltpu.VMEM((512,), jnp.float32),
            pltpu.SemaphoreType.DMA,
        ],
    )(xl2, xr2, att, src, dst)


def _passb_body(xl_ref, src_ref, dst_ref, w_ref, zn_ref, zd_ref,
                num_ref, den_ref,
                si_v, di_v, dri_v, rows_v, dr_v, w_v, num_acc, den_acc, sem):
    c = lax.axis_index("c")
    s = lax.axis_index("s")
    lane = lax.iota(jnp.int32, 16)
    pltpu.sync_copy(zn_ref, num_acc.at[pl.ds(s * NPS, NPS)])

    @pl.when((c == 0) & (s == 0))
    def _():
        pltpu.sync_copy(zd_ref, den_acc)

    plsc.subcore_barrier()

    def chunk(i, carry):
        base = s * EPS_B + i * CH
        pltpu.sync_copy(src_ref.at[pl.ds(base, CH)], si_v)
        pltpu.sync_copy(dst_ref.at[pl.ds(base, CH)], di_v)
        pltpu.sync_copy(w_ref.at[pl.ds(base, CH)], w_v)

        def adj(t, carry2):
            sl = pl.ds(t * 16, 16)
            si_v[sl] = si_v[sl] + c * NTOT
            dri_v[sl] = lax.shift_right_logical(di_v[sl], 4)
            return carry2
        lax.fori_loop(0, CH // 16, adj, 0)

        pltpu.async_copy(xl_ref.at[si_v], rows_v, sem).wait()

        def grp(t, carry2):
            wg = w_v[pl.ds(t * 16, 16)]
            for j in range(16):
                e = t * 16 + j
                ws = wg[j]
                for k in range(4):
                    sl = pl.ds(k * 16, 16)
                    rows_v[e, sl] = rows_v[e, sl] * ws
            return carry2
        lax.fori_loop(0, CH // 16, grp, 0)

        @pl.when(c == 0)
        def _():
            def dgrp(t, carry2):
                wg = w_v[pl.ds(t * 16, 16)]
                dmg = jnp.bitwise_and(di_v[pl.ds(t * 16, 16)], 15)
                for j in range(16):
                    dr_v[t * 16 + j, :] = jnp.where(
                        lane == dmg[j], wg[j], 0.0
                    )
                return carry2
            lax.fori_loop(0, CH // 16, dgrp, 0)

        pltpu.sync_copy(rows_v, num_acc.at[di_v], add=True)

        @pl.when(c == 0)
        def _():
            pltpu.sync_copy(dr_v, den_acc.at[dri_v], add=True)
        return carry
    lax.fori_loop(0, EPS_B // CH, chunk, 0)

    plsc.subcore_barrier()
    pltpu.sync_copy(num_acc.at[pl.ds(s * NPS, NPS)],
                    num_ref.at[pl.ds(c * NTOT + s * NPS, NPS)])

    @pl.when((c == 0) & (s == 0))
    def _():
        pltpu.sync_copy(den_acc, den_ref)


def _passb(xl2, src, dst, w, zn, zd):
    return pl.kernel(
        _passb_body,
        out_type=[
            jax.ShapeDtypeStruct((2 * NTOT, 64), jnp.float32),
            jax.ShapeDtypeStruct((NTOT // 16, 16), jnp.float32),
        ],
        mesh=_SC_MESH,
        compiler_params=pltpu.CompilerParams(use_tc_tiling_on_sc=False),
        scratch_types=[
            pltpu.VMEM((CH,), jnp.int32),
            pltpu.VMEM((CH,), jnp.int32),
            pltpu.VMEM((CH,), jnp.int32),
            pltpu.VMEM((CH, 64), jnp.float32),
            pltpu.VMEM((CH, 16), jnp.float32),
            pltpu.VMEM((CH,), jnp.float32),
            pltpu.VMEM_SHARED((NTOT, 64), jnp.float32),
            pltpu.VMEM_SHARED((NTOT // 16, 16), jnp.float32),
            pltpu.SemaphoreType.DMA,
        ],
    )(xl2, src, dst, w, zn, zd)


def _sc_edge_phase(xl2, xr2, att, src, dst, zn, zd):
    w = _passa(xl2, xr2, att, src, dst)
    num2, den = _passb(xl2, src, dst, w, zn, zd)
    return num2.reshape(2, NTOT, 64), den.reshape(NTOT, 1)


def kernel(x, edge_index, params):
    B, Nn, D = x.shape
    N = B * Nn
    src, dst = edge_index[0], edge_index[1]
    h0 = x.reshape(N, D)
    zn = jnp.zeros((NPS, 64), jnp.float32)
    zd = jnp.zeros((NPS, 16), jnp.float32)

    pa1 = params["a1"]
    xl = _mm_bias_split(h0, pa1["Wl"], pa1["bl"])
    xr = _mm_bias_split(h0, pa1["Wr"], pa1["br"])
    num, den = _sc_edge_phase(xl, xr, pa1["att"][0], src, dst, zn, zd)
    h_a = _combine(num, den, pa1["bias"], 0.01)

    pa2 = params["a2"]
    pad7 = 16 - NUM_CHOICES
    xl2 = _mm_bias16(
        h_a, jnp.pad(pa2["Wl"], ((0, 0), (0, pad7))), jnp.pad(pa2["bl"], (0, pad7))
    )
    xr2 = _mm_bias16(
        h_a, jnp.pad(pa2["Wr"], ((0, 0), (0, pad7))), jnp.pad(pa2["br"], (0, pad7))
    )
    att2 = jnp.pad(pa2["att"][0], (0, pad7))
    w2 = _passa2(xl2, xr2, att2, src, dst)
    num2, den2 = _passb2(xl2, src, dst, w2, zd)

    rows16 = lambda A: jnp.concatenate([A[0:NUM_COLUMNS], A[Nn : Nn + NUM_COLUMNS]], 0)
    numP = jnp.stack([rows16(num2[:NTOT]), rows16(num2[NTOT:])])
    den2f = den2.reshape(2, NTOT)
    denv = jnp.stack([rows16(den2f[0]), rows16(den2f[1])])
    denP = jnp.broadcast_to(denv[:, :, None], (2, 16, 16))
    numP = jnp.pad(numP, ((0, 0), (0, 0), (0, 112)))
    denP = jnp.pad(denP, ((0, 0), (0, 0), (0, 112)))
    b2p = jnp.pad(pa2["bias"], (0, 121)).reshape(1, 128)
    u = jax.random.uniform(
        jax.random.key(123), (B, NUM_COLUMNS, NUM_CHOICES), minval=1e-9, maxval=1.0
    )
    g = -jnp.log(-jnp.log(u))
    g16 = jnp.pad(g.reshape(16, NUM_CHOICES), ((0, 0), (0, 121)))
    act_p, sp_p = _sample(numP, denP, b2p, g16)
    action = act_p[:, :NUM_SELECTED].reshape(B, NUM_COLUMNS, NUM_SELECTED)
    selected_probs = sp_p[:, :NUM_SELECTED].reshape(B, NUM_COLUMNS, NUM_SELECTED)

    pc1 = params["c1"]
    cl = _mm_bias_split(h0, pc1["Wl"], pc1["bl"])
    cr = _mm_bias_split(h0, pc1["Wr"], pc1["br"])
    numc, denc = _sc_edge_phase(cl, cr, pc1["att"][0], src, dst, zn, zd)
    h_c = _combine(numc, denc, pc1["bias"], 0.01)

    pc2 = params["c2"]
    cl2 = _mm_bias_split(h_c, pc2["Wl"], pc2["bl"])
    cr2 = _mm_bias_split(h_c, pc2["Wr"], pc2["br"])
    numc2, denc2 = _sc_edge_phase(cl2, cr2, pc2["att"][0], src, dst, zn, zd)
    vfull = _value_head(numc2, denc2, pc2["bias"], params["fc_W"], params["fc_b"])
    value = vfull[:, :1].reshape(B, Nn, 1)

    return action, selected_probs, value
